# baseline (speedup 1.0000x reference)
import functools

import jax
import jax.numpy as jnp
from jax import lax
from jax.experimental import pallas as pl
from jax.experimental.pallas import tpu as pltpu
from jax.experimental.pallas import tpu_sc as plsc

_N = 10000
_E = 320000
_D = 128
_NC = 2
_NS = 16
_NW = _NC * _NS
_CH = 64
_EPW = 10240
_EPAD = _EPW * _NW
_NCHUNK = _EPW // _CH
_NPHASE = 4
_CPP = _NCHUNK // _NPHASE
_NBUF = 4
_NGRP = _CPP // _NBUF
_NPAD = 10240
_RPT_ACC = _NPAD // _NS
_DEGPAD = 10240
_RPT_DEG = _DEGPAD // _NS

_RB = 1000
_GRID = _N // _RB


def _sc_agg_body(with_deg, *refs):
    if with_deg:
        (x_hbm, src_hbm, dst_hbm, zrows_hbm, zdeg_hbm, acc_out, deg_out,
         src_v, dst_v, rows_v, ones_v, acc_sh, deg_sh, *sems) = refs
    else:
        (x_hbm, src_hbm, dst_hbm, zrows_hbm, acc_out,
         src_v, dst_v, rows_v, acc_sh, *sems) = refs
    gsems = sems[:_NBUF]
    ssems = sems[_NBUF:]
    c = lax.axis_index("c")
    s = lax.axis_index("s")
    wid = s * _NC + c

    pltpu.sync_copy(zrows_hbm, acc_sh.at[pl.ds(s * _RPT_ACC, _RPT_ACC), :])
    if with_deg:
        pltpu.sync_copy(zdeg_hbm, deg_sh.at[pl.ds(s * _RPT_DEG, _RPT_DEG)])
        for i in range(_CH // 16):
            ones_v[pl.ds(i * 16, 16)] = jnp.ones((16,), jnp.float32)
    plsc.subcore_barrier()

    def start_gather(g, slot):
        pltpu.async_copy(x_hbm.at[src_v.at[g]], rows_v.at[slot], gsems[slot])

    def wait_gather(g, slot):
        pltpu.make_async_copy(x_hbm.at[src_v.at[g]], rows_v.at[slot],
                              gsems[slot]).wait()

    def start_scatter(g, slot):
        pltpu.async_copy(rows_v.at[slot], acc_sh.at[dst_v.at[g]],
                         ssems[slot], add=True)
        if with_deg:
            pltpu.async_copy(ones_v, deg_sh.at[dst_v.at[g]], ssems[slot],
                             add=True)

    def wait_scatter(g, slot):
        pltpu.make_async_copy(rows_v.at[slot], acc_sh.at[dst_v.at[g]],
                              ssems[slot]).wait()
        if with_deg:
            pltpu.make_async_copy(ones_v, deg_sh.at[dst_v.at[g]],
                                  ssems[slot]).wait()

    def ring_group(j, _):
        for b in range(_NBUF):
            c = j * _NBUF + b
            wait_gather(c, b)
            start_scatter(c, b)
            nxt = c + _NBUF - 1
            b3 = (b + _NBUF - 1) % _NBUF

            @pl.when(nxt < _CPP)
            def _():
                @pl.when(c >= 1)
                def _():
                    wait_scatter(c - 1, b3)
                start_gather(nxt, b3)
        return 0

    for half in range(_NPHASE):
        pltpu.sync_copy(src_hbm.at[wid, pl.ds(half * _CPP, _CPP)], src_v)
        pltpu.sync_copy(dst_hbm.at[wid, pl.ds(half * _CPP, _CPP)], dst_v)
        for b in range(_NBUF - 1):
            start_gather(b, b)
        lax.fori_loop(0, _NGRP, ring_group, 0)
        for b in range(_NBUF):
            wait_scatter(_CPP - _NBUF + b, b)

    plsc.subcore_barrier()
    pltpu.sync_copy(acc_sh.at[pl.ds(s * _RPT_ACC, _RPT_ACC), :],
                    acc_out.at[c, pl.ds(s * _RPT_ACC, _RPT_ACC), :])
    if with_deg:
        pltpu.sync_copy(deg_sh.at[pl.ds(s * _RPT_DEG, _RPT_DEG)],
                        deg_out.at[c, pl.ds(s * _RPT_DEG, _RPT_DEG)])


@functools.lru_cache(maxsize=None)
def _make_sc_agg(with_deg):
    mesh = plsc.VectorSubcoreMesh(core_axis_name="c", subcore_axis_name="s",
                                  num_cores=_NC, num_subcores=_NS)
    out_type = [jax.ShapeDtypeStruct((_NC, _NPAD, _D), jnp.float32)]
    scratch = [
        pltpu.VMEM((_CPP, _CH), jnp.int32),
        pltpu.VMEM((_CPP, _CH), jnp.int32),
        pltpu.VMEM((_NBUF, _CH, _D), jnp.float32),
    ]
    if with_deg:
        out_type.append(jax.ShapeDtypeStruct((_NC, _DEGPAD), jnp.float32))
        scratch.append(pltpu.VMEM((_CH,), jnp.float32))
    scratch.append(pltpu.VMEM_SHARED((_NPAD, _D), jnp.float32))
    if with_deg:
        scratch.append(pltpu.VMEM_SHARED((_DEGPAD,), jnp.float32))
    for _ in range(2 * _NBUF):
        scratch.append(pltpu.SemaphoreType.DMA)
    return pl.kernel(
        functools.partial(_sc_agg_body, with_deg),
        out_type=out_type,
        mesh=mesh,
        scratch_types=scratch,
    )


def _dense_body(with_relu, emit_recip, x_ref, acc_ref, deg_ref, ws_ref,
                wn_ref, b_ref, *out_refs):
    if emit_recip:
        d = jnp.maximum(deg_ref[0] + deg_ref[1], 1.0)
        r = 1.0 / d
    else:
        r = deg_ref[...]
    hn = (acc_ref[0] + acc_ref[1]) * r
    y = (jnp.dot(x_ref[...], ws_ref[...], preferred_element_type=jnp.float32)
         + jnp.dot(hn, wn_ref[...], preferred_element_type=jnp.float32)
         + b_ref[...])
    if with_relu:
        y = jnp.maximum(y, 0.0)
    out_refs[0][...] = y
    if emit_recip:
        out_refs[1][...] = r


def _make_dense(with_relu, emit_recip):
    deg_spec = (pl.BlockSpec((2, _RB, 1), lambda i: (0, i, 0)) if emit_recip
                else pl.BlockSpec((_RB, 1), lambda i: (i, 0)))
    out_shape = [jax.ShapeDtypeStruct((_N, _D), jnp.float32)]
    out_specs = [pl.BlockSpec((_RB, _D), lambda i: (i, 0))]
    if emit_recip:
        out_shape.append(jax.ShapeDtypeStruct((_N, 1), jnp.float32))
        out_specs.append(pl.BlockSpec((_RB, 1), lambda i: (i, 0)))
    return pl.pallas_call(
        functools.partial(_dense_body, with_relu, emit_recip),
        grid=(_GRID,),
        in_specs=[
            pl.BlockSpec((_RB, _D), lambda i: (i, 0)),
            pl.BlockSpec((2, _RB, _D), lambda i: (0, i, 0)),
            deg_spec,
            pl.BlockSpec((_D, _D), lambda i: (0, 0)),
            pl.BlockSpec((_D, _D), lambda i: (0, 0)),
            pl.BlockSpec((1, _D), lambda i: (0, 0)),
        ],
        out_specs=out_specs,
        out_shape=out_shape,
    )


_dense1 = _make_dense(True, True)
_dense2 = _make_dense(False, False)


def kernel(h, edge_index, W_self1, W_neigh1, b1, W_self2, W_neigh2, b2):
    edges = edge_index.astype(jnp.int32)
    pad_n = _EPAD - _E
    pad_ar = jnp.arange(pad_n, dtype=jnp.int32)
    src = jnp.concatenate([edges[0], pad_ar % _N]).reshape(_NW, _NCHUNK, _CH)
    dst = jnp.concatenate([edges[1], _N + pad_ar % (_NPAD - _N)]
                          ).reshape(_NW, _NCHUNK, _CH)
    zrows = jnp.zeros((_RPT_ACC, _D), jnp.float32)
    zdeg = jnp.zeros((_RPT_DEG,), jnp.float32)

    acc1, deg = _make_sc_agg(True)(h, src, dst, zrows, zdeg)
    deg3 = deg.reshape(_NC, _DEGPAD, 1)
    x, recip = _dense1(h, acc1, deg3, W_self1, W_neigh1, b1.reshape(1, _D))
    (acc2,) = _make_sc_agg(False)(x, src, dst, zrows)
    (out,) = _dense2(x, acc2, recip, W_self2, W_neigh2, b2.reshape(1, _D))
    return out


# speedup vs baseline: 1.0534x; 1.0534x over previous
import functools

import jax
import jax.numpy as jnp
from jax import lax
from jax.experimental import pallas as pl
from jax.experimental.pallas import tpu as pltpu
from jax.experimental.pallas import tpu_sc as plsc

_N = 10000
_E = 320000
_D = 128
_NC = 2
_NS = 16
_NW = _NC * _NS
_CH = 128
_EPW = 10240
_EPAD = _EPW * _NW
_NCHUNK = _EPW // _CH
_NPHASE = 2
_CPP = _NCHUNK // _NPHASE
_NPAIR = _CPP // 2
_NPAD = 10240
_RPT_ACC = _NPAD // _NS
_DEGPAD = 10240
_RPT_DEG = _DEGPAD // _NS

_RB = 1000
_GRID = _N // _RB


def _sc_agg_body(with_deg, *refs):
    if with_deg:
        (x_hbm, e_hbm, zrows_hbm, zdeg_hbm, acc_out, deg_out,
         src_v, dst_v, rows_v, ones_v, acc_sh, deg_sh, sem0, sem1) = refs
    else:
        (x_hbm, e_hbm, zrows_hbm, acc_out,
         src_v, dst_v, rows_v, acc_sh, sem0, sem1) = refs
    c = lax.axis_index("c")
    s = lax.axis_index("s")
    wid = s * _NC + c

    pltpu.sync_copy(zrows_hbm, acc_sh.at[pl.ds(s * _RPT_ACC, _RPT_ACC), :])
    if with_deg:
        pltpu.sync_copy(zdeg_hbm, deg_sh.at[pl.ds(s * _RPT_DEG, _RPT_DEG)])
        for i in range(_CH // 16):
            ones_v[pl.ds(i * 16, 16)] = jnp.ones((16,), jnp.float32)
    plsc.subcore_barrier()

    sems = (sem0, sem1)

    def start_gather(g, slot):
        pltpu.async_copy(x_hbm.at[src_v.at[g]], rows_v.at[slot], sems[slot])

    def drain_and_scatter(g, slot):
        pltpu.make_async_copy(x_hbm.at[src_v.at[g]], rows_v.at[slot],
                              sems[slot]).wait()
        pltpu.sync_copy(rows_v.at[slot], acc_sh.at[dst_v.at[g]], add=True)
        if with_deg:
            pltpu.sync_copy(ones_v, deg_sh.at[dst_v.at[g]], add=True)

    def step_pair(p, _):
        g0 = 2 * p
        start_gather(g0 + 1, 1)
        drain_and_scatter(g0, 0)

        @pl.when(p < _NPAIR - 1)
        def _():
            start_gather(g0 + 2, 0)

        drain_and_scatter(g0 + 1, 1)
        return 0

    for half in range(_NPHASE):
        pltpu.sync_copy(e_hbm.at[0, wid, pl.ds(half * _CPP, _CPP)], src_v)
        pltpu.sync_copy(e_hbm.at[1, wid, pl.ds(half * _CPP, _CPP)], dst_v)
        start_gather(0, 0)
        lax.fori_loop(0, _NPAIR, step_pair, 0)

    plsc.subcore_barrier()
    pltpu.sync_copy(acc_sh.at[pl.ds(s * _RPT_ACC, _RPT_ACC), :],
                    acc_out.at[c, pl.ds(s * _RPT_ACC, _RPT_ACC), :])
    if with_deg:
        pltpu.sync_copy(deg_sh.at[pl.ds(s * _RPT_DEG, _RPT_DEG)],
                        deg_out.at[c, pl.ds(s * _RPT_DEG, _RPT_DEG)])


@functools.lru_cache(maxsize=None)
def _make_sc_agg(with_deg):
    mesh = plsc.VectorSubcoreMesh(core_axis_name="c", subcore_axis_name="s",
                                  num_cores=_NC, num_subcores=_NS)
    out_type = [jax.ShapeDtypeStruct((_NC, _NPAD, _D), jnp.float32)]
    scratch = [
        pltpu.VMEM((_CPP, _CH), jnp.int32),
        pltpu.VMEM((_CPP, _CH), jnp.int32),
        pltpu.VMEM((2, _CH, _D), jnp.float32),
    ]
    if with_deg:
        out_type.append(jax.ShapeDtypeStruct((_NC, _DEGPAD), jnp.float32))
        scratch.append(pltpu.VMEM((_CH,), jnp.float32))
    scratch.append(pltpu.VMEM_SHARED((_NPAD, _D), jnp.float32))
    if with_deg:
        scratch.append(pltpu.VMEM_SHARED((_DEGPAD,), jnp.float32))
    scratch.append(pltpu.SemaphoreType.DMA)
    scratch.append(pltpu.SemaphoreType.DMA)
    return pl.kernel(
        functools.partial(_sc_agg_body, with_deg),
        out_type=out_type,
        mesh=mesh,
        scratch_types=scratch,
    )


def _dense_body(with_relu, emit_recip, x_ref, acc_ref, deg_ref, ws_ref,
                wn_ref, b_ref, *out_refs):
    if emit_recip:
        d = jnp.maximum(deg_ref[0] + deg_ref[1], 1.0)
        r = 1.0 / d
    else:
        r = deg_ref[...]
    hn = (acc_ref[0] + acc_ref[1]) * r
    y = (jnp.dot(x_ref[...], ws_ref[...], preferred_element_type=jnp.float32)
         + jnp.dot(hn, wn_ref[...], preferred_element_type=jnp.float32)
         + b_ref[...])
    if with_relu:
        y = jnp.maximum(y, 0.0)
    out_refs[0][...] = y
    if emit_recip:
        out_refs[1][...] = r


def _make_dense(with_relu, emit_recip):
    deg_spec = (pl.BlockSpec((2, _RB, 1), lambda i: (0, i, 0)) if emit_recip
                else pl.BlockSpec((_RB, 1), lambda i: (i, 0)))
    out_shape = [jax.ShapeDtypeStruct((_N, _D), jnp.float32)]
    out_specs = [pl.BlockSpec((_RB, _D), lambda i: (i, 0))]
    if emit_recip:
        out_shape.append(jax.ShapeDtypeStruct((_N, 1), jnp.float32))
        out_specs.append(pl.BlockSpec((_RB, 1), lambda i: (i, 0)))
    return pl.pallas_call(
        functools.partial(_dense_body, with_relu, emit_recip),
        grid=(_GRID,),
        in_specs=[
            pl.BlockSpec((_RB, _D), lambda i: (i, 0)),
            pl.BlockSpec((2, _RB, _D), lambda i: (0, i, 0)),
            deg_spec,
            pl.BlockSpec((_D, _D), lambda i: (0, 0)),
            pl.BlockSpec((_D, _D), lambda i: (0, 0)),
            pl.BlockSpec((1, _D), lambda i: (0, 0)),
        ],
        out_specs=out_specs,
        out_shape=out_shape,
    )


_dense1 = _make_dense(True, True)
_dense2 = _make_dense(False, False)


def kernel(h, edge_index, W_self1, W_neigh1, b1, W_self2, W_neigh2, b2):
    edges = edge_index.astype(jnp.int32)
    pad_n = _EPAD - _E
    pad_ar = jnp.arange(pad_n, dtype=jnp.int32)
    pad_blk = jnp.stack([pad_ar % _N, _N + pad_ar % (_NPAD - _N)])
    e4 = jnp.concatenate([edges, pad_blk], axis=1).reshape(
        2, _NW, _NCHUNK, _CH)
    zrows = jnp.zeros((_RPT_ACC, _D), jnp.float32)
    zdeg = jnp.zeros((_RPT_DEG,), jnp.float32)

    acc1, deg = _make_sc_agg(True)(h, e4, zrows, zdeg)
    deg3 = deg.reshape(_NC, _DEGPAD, 1)
    x, recip = _dense1(h, acc1, deg3, W_self1, W_neigh1, b1.reshape(1, _D))
    (acc2,) = _make_sc_agg(False)(x, e4, zrows)
    (out,) = _dense2(x, acc2, recip, W_self2, W_neigh2, b2.reshape(1, _D))
    return out
